# baseline (speedup 1.0000x reference)
import jax
import jax.numpy as jnp
from jax.experimental import pallas as pl
from jax.experimental.pallas import tpu as pltpu

NUM_TIMESTEPS = 1000
BETA_START = 0.0001
BETA_END = 0.02

_ROWS = 8


def _body(t_ref, sac_ref, somac_ref, x_ref, n_ref, o_ref):
    base = pl.program_id(0) * _ROWS
    for r in range(_ROWS):
        tt = t_ref[base + r]
        o_ref[r] = sac_ref[tt] * x_ref[r] + somac_ref[tt] * n_ref[r]


def _tables():
    betas = jnp.linspace(BETA_START, BETA_END, NUM_TIMESTEPS, dtype=jnp.float32)
    alphas_cumprod = jnp.cumprod(1.0 - betas, axis=0)
    sac = jnp.sqrt(alphas_cumprod)
    somac = jnp.sqrt(1.0 - alphas_cumprod)
    return sac, somac


def kernel(x_start, t, noise):
    B = x_start.shape[0]
    F = x_start.size // B
    S = F // 128
    x = x_start.reshape(B, S, 128)
    n = noise.reshape(B, S, 128)
    sac, somac = _tables()
    t32 = t.astype(jnp.int32)

    out = pl.pallas_call(
        _body,
        grid_spec=pltpu.PrefetchScalarGridSpec(
            num_scalar_prefetch=3,
            grid=(B // _ROWS,),
            in_specs=[
                pl.BlockSpec((_ROWS, S, 128), lambda i, *_: (i, 0, 0)),
                pl.BlockSpec((_ROWS, S, 128), lambda i, *_: (i, 0, 0)),
            ],
            out_specs=pl.BlockSpec((_ROWS, S, 128), lambda i, *_: (i, 0, 0)),
        ),
        out_shape=jax.ShapeDtypeStruct((B, S, 128), jnp.float32),
    )(t32, sac, somac, x, n)
    return out.reshape(x_start.shape)


# speedup vs baseline: 1.1708x; 1.1708x over previous
import jax
import jax.numpy as jnp
from jax.experimental import pallas as pl
from jax.experimental.pallas import tpu as pltpu

NUM_TIMESTEPS = 1000
BETA_START = 0.0001
BETA_END = 0.02

_ROWS = 64


def _body(t_ref, sac_ref, somac_ref, x_ref, n_ref, o_ref):
    base = pl.program_id(0) * _ROWS
    for r in range(_ROWS):
        tt = t_ref[base + r]
        o_ref[r] = sac_ref[tt] * x_ref[r] + somac_ref[tt] * n_ref[r]


def _tables():
    betas = jnp.linspace(BETA_START, BETA_END, NUM_TIMESTEPS, dtype=jnp.float32)
    alphas_cumprod = jnp.cumprod(1.0 - betas, axis=0)
    sac = jnp.sqrt(alphas_cumprod)
    somac = jnp.sqrt(1.0 - alphas_cumprod)
    return sac, somac


def kernel(x_start, t, noise):
    B = x_start.shape[0]
    F = x_start.size // B
    S = F // 128
    x = x_start.reshape(B, S, 128)
    n = noise.reshape(B, S, 128)
    sac, somac = _tables()
    t32 = t.astype(jnp.int32)

    out = pl.pallas_call(
        _body,
        grid_spec=pltpu.PrefetchScalarGridSpec(
            num_scalar_prefetch=3,
            grid=(B // _ROWS,),
            in_specs=[
                pl.BlockSpec((_ROWS, S, 128), lambda i, *_: (i, 0, 0)),
                pl.BlockSpec((_ROWS, S, 128), lambda i, *_: (i, 0, 0)),
            ],
            out_specs=pl.BlockSpec((_ROWS, S, 128), lambda i, *_: (i, 0, 0)),
        ),
        out_shape=jax.ShapeDtypeStruct((B, S, 128), jnp.float32),
    )(t32, sac, somac, x, n)
    return out.reshape(x_start.shape)
